# baseline (speedup 1.0000x reference)
import functools
import math

import jax
import jax.numpy as jnp
from jax import lax
from jax.experimental import pallas as pl
from jax.experimental.pallas import tpu as pltpu
from jax.experimental.pallas import tpu_sc as plsc

D = 64
LANES = 16
CH = 256
IDX_BLK = 128
NC = 2
NS = 16
NW = NC * NS
S = 512


def _sc_body(scale, n_rows, tok_hbm, x_hbm, seg_hbm, comb_hbm, out_hbm,
             xidx_all, p2, d1, toka, tokb, pba, pbb, sega, segb,
             gsem_a, gsem_b, ssem_a, ssem_b):
    wid = lax.axis_index("s") * NC + lax.axis_index("c")
    rows_per_w = n_rows // NW
    n_chunks = rows_per_w // CH
    n_half = n_chunks // 2
    idx_rows = rows_per_w // IDX_BLK
    blk = CH // IDX_BLK
    iota = lax.iota(jnp.int32, LANES)

    pltpu.sync_copy(x_hbm.at[pl.ds(wid * idx_rows, idx_rows)], xidx_all)
    pltpu.sync_copy(comb_hbm.at[pl.ds(0, S)], p2)
    pltpu.sync_copy(comb_hbm.at[pl.ds(S, 1)], d1)
    delta = [d1[0, pl.ds(k * LANES, LANES)] - p2[0, pl.ds(k * LANES, LANES)]
             for k in range(D // LANES)]

    def start_gather(c, tokbuf, segbuf, gsem):
        for j in range(blk):
            pltpu.make_async_copy(
                tok_hbm.at[xidx_all.at[blk * c + j]],
                tokbuf.at[pl.ds(j * IDX_BLK, IDX_BLK)], gsem).start()
        pltpu.make_async_copy(
            seg_hbm.at[pl.ds(wid * rows_per_w + c * CH, CH)], segbuf,
            gsem).start()

    def wait_gather(tokbuf, segbuf, gsem):
        pltpu.make_async_copy(tok_hbm.at[pl.ds(0, CH)], tokbuf, gsem).wait()
        pltpu.make_async_copy(seg_hbm.at[pl.ds(0, CH)], segbuf, gsem).wait()

    def combine(c, tokbuf, segbuf, pbuf):
        pos0 = lax.rem(c * CH, S)

        @pl.loop(0, CH // LANES)
        def _grp(g):
            r0 = g * LANES
            sv = segbuf[pl.ds(r0, LANES)].astype(jnp.float32)
            for i in range(LANES):
                r = r0 + i
                sf = sv[i]
                col = iota * 0 + r
                for k in range(D // LANES):
                    sl = pl.ds(k * LANES, LANES)
                    v = (tokbuf[r, sl] * scale
                         + (p2[pos0 + r, sl] + sf * delta[k]))
                    plsc.store_scatter(pbuf, [k * LANES + iota, col], v)

    def out_slice(c):
        gr0 = wid * rows_per_w + c * CH
        b = lax.div(gr0, S)
        s0 = lax.rem(gr0, S)
        return out_hbm.at[b, :, pl.ds(s0, CH)]

    start_gather(0, toka, sega, gsem_a)

    @pl.loop(0, n_half)
    def _pipe(t):
        c0 = 2 * t

        @pl.when(t > 0)
        def _free_b():
            pltpu.make_async_copy(pbb, out_slice(c0 - 1), ssem_b).wait()

        start_gather(c0 + 1, tokb, segb, gsem_b)

        wait_gather(toka, sega, gsem_a)
        combine(c0, toka, sega, pba)
        pltpu.make_async_copy(pba, out_slice(c0), ssem_a).start()

        @pl.when(t + 1 < n_half)
        def _next_a():
            pltpu.make_async_copy(pba, out_slice(c0), ssem_a).wait()
            start_gather(c0 + 2, toka, sega, gsem_a)

        wait_gather(tokb, segb, gsem_b)
        combine(c0 + 1, tokb, segb, pbb)
        pltpu.make_async_copy(pbb, out_slice(c0 + 1), ssem_b).start()

    pltpu.make_async_copy(pba, out_slice(n_chunks - 2), ssem_a).wait()
    pltpu.make_async_copy(pbb, out_slice(n_chunks - 1), ssem_b).wait()


@functools.partial(jax.jit, static_argnames=("n_rows",))
def _sc_embed(token_table, x_idx, seg_flat, comb, n_rows):
    scale = float(math.sqrt(D))
    mesh = plsc.VectorSubcoreMesh(core_axis_name="c", subcore_axis_name="s")
    idx_rows = n_rows // NW // IDX_BLK
    grid_kernel = pl.kernel(
        functools.partial(_sc_body, scale, n_rows),
        out_type=jax.ShapeDtypeStruct((n_rows // S, D, S), jnp.float32),
        mesh=mesh,
        compiler_params=pltpu.CompilerParams(use_tc_tiling_on_sc=False,
                                             needs_layout_passes=False),
        scratch_types=[
            pltpu.VMEM((idx_rows, IDX_BLK), jnp.int32),
            pltpu.VMEM((S, D), jnp.float32),
            pltpu.VMEM((1, D), jnp.float32),
            pltpu.VMEM((CH, D), jnp.float32),
            pltpu.VMEM((CH, D), jnp.float32),
            pltpu.VMEM((D, CH), jnp.float32),
            pltpu.VMEM((D, CH), jnp.float32),
            pltpu.VMEM((CH,), jnp.int32),
            pltpu.VMEM((CH,), jnp.int32),
            pltpu.SemaphoreType.DMA,
            pltpu.SemaphoreType.DMA,
            pltpu.SemaphoreType.DMA,
            pltpu.SemaphoreType.DMA,
        ],
    )
    return grid_kernel(token_table, x_idx, seg_flat, comb)


def kernel(x, segment_info, token_table, pos_embedding, segment_table):
    B, S_in = x.shape
    n_rows = B * S_in
    assert S_in == S and n_rows % (NW * CH) == 0 and CH % IDX_BLK == 0
    x_idx = x.reshape(n_rows // IDX_BLK, IDX_BLK).astype(jnp.int32)
    seg_flat = segment_info.reshape(n_rows).astype(jnp.int32)
    comb = (pos_embedding[0, :S_in, :][None, :, :]
            + segment_table[:, None, :]).reshape(-1, D)
    out_t = _sc_embed(token_table, x_idx, seg_flat, comb, n_rows)
    return jnp.transpose(out_t, (0, 2, 1))


# speedup vs baseline: 2.0669x; 2.0669x over previous
import functools
import math

import jax
import jax.numpy as jnp
from jax import lax
from jax.experimental import pallas as pl
from jax.experimental.pallas import tpu as pltpu
from jax.experimental.pallas import tpu_sc as plsc

D = 64
LANES = 16
CH = 512
IDX_BLK = 128
NC = 2
NS = 16
NW = NC * NS
S = 512


def _sc_body(scale, n_rows, tok_hbm, x_hbm, seg_hbm, comb_hbm, out_hbm,
             xidx_all, p2, d1, toka, tokb, sega, segb,
             gsem_a, gsem_b, ssem_a, ssem_b):
    wid = lax.axis_index("s") * NC + lax.axis_index("c")
    rows_per_w = n_rows // NW
    n_chunks = rows_per_w // CH
    n_half = n_chunks // 2
    idx_rows = rows_per_w // IDX_BLK
    blk = CH // IDX_BLK

    pltpu.sync_copy(x_hbm.at[pl.ds(wid * idx_rows, idx_rows)], xidx_all)
    pltpu.sync_copy(comb_hbm.at[pl.ds(0, S)], p2)
    pltpu.sync_copy(comb_hbm.at[pl.ds(S, 1)], d1)
    delta = [d1[0, pl.ds(k * LANES, LANES)] - p2[0, pl.ds(k * LANES, LANES)]
             for k in range(D // LANES)]

    def start_gather(c, tokbuf, segbuf, gsem):
        for j in range(blk):
            pltpu.make_async_copy(
                tok_hbm.at[xidx_all.at[blk * c + j]],
                tokbuf.at[pl.ds(j * IDX_BLK, IDX_BLK)], gsem).start()
        pltpu.make_async_copy(
            seg_hbm.at[pl.ds(wid * rows_per_w + c * CH, CH)], segbuf,
            gsem).start()

    def wait_gather(tokbuf, segbuf, gsem):
        pltpu.make_async_copy(tok_hbm.at[pl.ds(0, CH)], tokbuf, gsem).wait()
        pltpu.make_async_copy(seg_hbm.at[pl.ds(0, CH)], segbuf, gsem).wait()

    def fma(tokbuf, segbuf):
        @pl.loop(0, CH // LANES)
        def _grp(g):
            r0 = g * LANES
            sv = segbuf[pl.ds(r0, LANES)].astype(jnp.float32)
            for i in range(LANES):
                r = r0 + i
                sf = sv[i]
                for k in range(D // LANES):
                    sl = pl.ds(k * LANES, LANES)
                    tokbuf[r, sl] = (tokbuf[r, sl] * scale
                                     + (p2[r, sl] + sf * delta[k]))

    def out_slice(c):
        return out_hbm.at[pl.ds(wid * rows_per_w + c * CH, CH)]

    start_gather(0, toka, sega, gsem_a)

    @pl.loop(0, n_half)
    def _pipe(t):
        c0 = 2 * t

        @pl.when(t > 0)
        def _free_b():
            pltpu.make_async_copy(tokb, out_slice(c0 - 1), ssem_b).wait()

        start_gather(c0 + 1, tokb, segb, gsem_b)

        wait_gather(toka, sega, gsem_a)
        fma(toka, sega)
        pltpu.make_async_copy(toka, out_slice(c0), ssem_a).start()

        @pl.when(t + 1 < n_half)
        def _next_a():
            pltpu.make_async_copy(toka, out_slice(c0), ssem_a).wait()
            start_gather(c0 + 2, toka, sega, gsem_a)

        wait_gather(tokb, segb, gsem_b)
        fma(tokb, segb)
        pltpu.make_async_copy(tokb, out_slice(c0 + 1), ssem_b).start()

    pltpu.make_async_copy(toka, out_slice(n_chunks - 2), ssem_a).wait()
    pltpu.make_async_copy(tokb, out_slice(n_chunks - 1), ssem_b).wait()


@functools.partial(jax.jit, static_argnames=("n_rows",))
def _sc_embed(token_table, x_idx, seg_flat, comb, n_rows):
    scale = float(math.sqrt(D))
    mesh = plsc.VectorSubcoreMesh(core_axis_name="c", subcore_axis_name="s")
    idx_rows = n_rows // NW // IDX_BLK
    grid_kernel = pl.kernel(
        functools.partial(_sc_body, scale, n_rows),
        out_type=jax.ShapeDtypeStruct((n_rows, D), jnp.float32),
        mesh=mesh,
        compiler_params=pltpu.CompilerParams(use_tc_tiling_on_sc=False),
        scratch_types=[
            pltpu.VMEM((idx_rows, IDX_BLK), jnp.int32),
            pltpu.VMEM((S, D), jnp.float32),
            pltpu.VMEM((1, D), jnp.float32),
            pltpu.VMEM((CH, D), jnp.float32),
            pltpu.VMEM((CH, D), jnp.float32),
            pltpu.VMEM((CH,), jnp.int32),
            pltpu.VMEM((CH,), jnp.int32),
            pltpu.SemaphoreType.DMA,
            pltpu.SemaphoreType.DMA,
            pltpu.SemaphoreType.DMA,
            pltpu.SemaphoreType.DMA,
        ],
    )
    return grid_kernel(token_table, x_idx, seg_flat, comb)


def kernel(x, segment_info, token_table, pos_embedding, segment_table):
    B, S_in = x.shape
    n_rows = B * S_in
    assert S_in == S and n_rows % (NW * CH) == 0 and CH % IDX_BLK == 0
    x_idx = x.reshape(n_rows // IDX_BLK, IDX_BLK).astype(jnp.int32)
    seg_flat = segment_info.reshape(n_rows).astype(jnp.int32)
    comb = (pos_embedding[0, :S_in, :][None, :, :]
            + segment_table[:, None, :]).reshape(-1, D)
    out = _sc_embed(token_table, x_idx, seg_flat, comb, n_rows)
    return out.reshape(B, S_in, D)


# speedup vs baseline: 2.2268x; 1.0774x over previous
import functools
import math

import jax
import jax.numpy as jnp
from jax import lax
from jax.experimental import pallas as pl
from jax.experimental.pallas import tpu as pltpu
from jax.experimental.pallas import tpu_sc as plsc

D = 64
LANES = 16
CH = 256
NBUF = 4
IDX_BLK = 128
NC = 2
NS = 16
NW = NC * NS
S = 512


def _sc_body(scale, n_rows, tok_hbm, x_hbm, seg_hbm, comb_hbm, out_hbm,
             xidx_all, p2, d1,
             tok0, tok1, tok2, tok3, seg0, seg1, seg2, seg3,
             gs0, gs1, gs2, gs3, ss0, ss1, ss2, ss3):
    wid = lax.axis_index("s") * NC + lax.axis_index("c")
    rows_per_w = n_rows // NW
    n_chunks = rows_per_w // CH
    idx_rows = rows_per_w // IDX_BLK
    blk = CH // IDX_BLK
    bufs = [(tok0, seg0, gs0, ss0), (tok1, seg1, gs1, ss1),
            (tok2, seg2, gs2, ss2), (tok3, seg3, gs3, ss3)]

    pltpu.sync_copy(x_hbm.at[pl.ds(wid * idx_rows, idx_rows)], xidx_all)
    pltpu.sync_copy(comb_hbm.at[pl.ds(0, S)], p2)
    pltpu.sync_copy(comb_hbm.at[pl.ds(S, 1)], d1)
    delta = [d1[0, pl.ds(k * LANES, LANES)] - p2[0, pl.ds(k * LANES, LANES)]
             for k in range(D // LANES)]

    def start_gather(c, tokbuf, segbuf, gsem):
        for j in range(blk):
            pltpu.make_async_copy(
                tok_hbm.at[xidx_all.at[blk * c + j]],
                tokbuf.at[pl.ds(j * IDX_BLK, IDX_BLK)], gsem).start()
        pltpu.make_async_copy(
            seg_hbm.at[pl.ds(wid * rows_per_w + c * CH, CH)], segbuf,
            gsem).start()

    def wait_gather(tokbuf, segbuf, gsem):
        pltpu.make_async_copy(tok_hbm.at[pl.ds(0, CH)], tokbuf, gsem).wait()
        pltpu.make_async_copy(seg_hbm.at[pl.ds(0, CH)], segbuf, gsem).wait()

    def fma(c, tokbuf, segbuf):
        pos0 = lax.rem(c * CH, S)

        @pl.loop(0, CH // LANES)
        def _grp(g):
            r0 = g * LANES
            sv = segbuf[pl.ds(r0, LANES)].astype(jnp.float32)
            for i in range(LANES):
                r = r0 + i
                sf = sv[i]
                for k in range(D // LANES):
                    sl = pl.ds(k * LANES, LANES)
                    tokbuf[r, sl] = (tokbuf[r, sl] * scale
                                     + (p2[pos0 + r, sl] + sf * delta[k]))

    def out_slice(c):
        return out_hbm.at[pl.ds(wid * rows_per_w + c * CH, CH)]

    for u in range(NBUF - 1):
        start_gather(u, bufs[u][0], bufs[u][1], bufs[u][2])

    @pl.loop(0, n_chunks // NBUF)
    def _pipe(t):
        for u in range(NBUF):
            tokbuf, segbuf, gsem, ssem = bufs[u]
            c = NBUF * t + u
            wait_gather(tokbuf, segbuf, gsem)
            fma(c, tokbuf, segbuf)
            pltpu.make_async_copy(tokbuf, out_slice(c), ssem).start()

            tv, sv_, gv, sw = bufs[(u + NBUF - 1) % NBUF]

            @pl.when(c + NBUF - 1 < n_chunks)
            def _nx():
                @pl.when(c >= 1)
                def _w():
                    pltpu.make_async_copy(tv, out_slice(c - 1), sw).wait()

                start_gather(c + NBUF - 1, tv, sv_, gv)

    for u in range(NBUF):
        c = n_chunks - NBUF + u
        pltpu.make_async_copy(bufs[u][0], out_slice(c), bufs[u][3]).wait()


@functools.partial(jax.jit, static_argnames=("n_rows",))
def _sc_embed(token_table, x_idx, seg_flat, comb, n_rows):
    scale = float(math.sqrt(D))
    mesh = plsc.VectorSubcoreMesh(core_axis_name="c", subcore_axis_name="s")
    idx_rows = n_rows // NW // IDX_BLK
    grid_kernel = pl.kernel(
        functools.partial(_sc_body, scale, n_rows),
        out_type=jax.ShapeDtypeStruct((n_rows, D), jnp.float32),
        mesh=mesh,
        compiler_params=pltpu.CompilerParams(use_tc_tiling_on_sc=False),
        scratch_types=(
            [pltpu.VMEM((idx_rows, IDX_BLK), jnp.int32),
             pltpu.VMEM((S, D), jnp.float32),
             pltpu.VMEM((1, D), jnp.float32)]
            + [pltpu.VMEM((CH, D), jnp.float32)] * NBUF
            + [pltpu.VMEM((CH,), jnp.int32)] * NBUF
            + [pltpu.SemaphoreType.DMA] * (2 * NBUF)
        ),
    )
    return grid_kernel(token_table, x_idx, seg_flat, comb)


def kernel(x, segment_info, token_table, pos_embedding, segment_table):
    B, S_in = x.shape
    n_rows = B * S_in
    assert S_in == S and n_rows % (NW * NBUF * CH) == 0 and CH % IDX_BLK == 0
    x_idx = x.reshape(n_rows // IDX_BLK, IDX_BLK).astype(jnp.int32)
    seg_flat = segment_info.reshape(n_rows).astype(jnp.int32)
    comb = (pos_embedding[0, :S_in, :][None, :, :]
            + segment_table[:, None, :]).reshape(-1, D)
    out = _sc_embed(token_table, x_idx, seg_flat, comb, n_rows)
    return out.reshape(B, S_in, D)
